# baseline (speedup 1.0000x reference)
import functools

import jax
import jax.numpy as jnp
from jax import lax
from jax.experimental import pallas as pl
from jax.experimental.pallas import tpu as pltpu
from jax.experimental.pallas import tpu_sc as plsc

D = 64
L = 16
CHUNK = 128
NBUF = 3


def _rsqrt(x):
    i = plsc.bitcast(x, jnp.int32)
    i = jnp.int32(0x5F3759DF) - (i >> 1)
    y = plsc.bitcast(i, jnp.float32)
    h = x * jnp.float32(0.5)
    for _ in range(2):
        y = y * (jnp.float32(1.5) - h * y * y)
    return y


@functools.partial(jax.jit, static_argnames=("b_total",))
def _embed_normalize(x2, table2, b_total):
    info = plsc.get_sparse_core_info()
    nc, ns = info.num_cores, info.num_subcores
    nw = nc * ns
    b_per_w = b_total // nw
    n_chunks = b_per_w // CHUNK
    xrows_per_w = b_per_w // 128
    orows_per_w = b_per_w // 2
    mesh = plsc.VectorSubcoreMesh(core_axis_name="c", subcore_axis_name="s")

    @functools.partial(
        pl.kernel,
        mesh=mesh,
        out_type=jax.ShapeDtypeStruct((b_total // 2, 128), jnp.float32),
        compiler_params=pltpu.CompilerParams(needs_layout_passes=False),
        scratch_types=[
            pltpu.VMEM((NBUF, 8, 128), jnp.int32),
            pltpu.VMEM((NBUF, 128), jnp.int32),
            pltpu.VMEM((NBUF, CHUNK, 128), jnp.float32),
            pltpu.VMEM((NBUF, CHUNK // 2, 128), jnp.float32),
            pltpu.SemaphoreType.DMA((NBUF,)),
            pltpu.SemaphoreType.DMA((NBUF,)),
            pltpu.SemaphoreType.DMA((NBUF,)),
        ],
    )
    def body(x_hbm, table_hbm, out_hbm,
             xb_v, fb_v, gb_v, ob_v, sem_x, sem_g, sem_o):
        wid = lax.axis_index("s") * nc + lax.axis_index("c")
        xbase = wid * xrows_per_w
        obase = wid * orows_per_w

        def slot(g):
            return lax.rem(g, NBUF)

        def x_dma(g):
            b = slot(g)
            ab = ((xbase + g) // 8) * 8
            return pltpu.make_async_copy(
                x_hbm.at[pl.ds(ab, 8)], xb_v.at[b], sem_x.at[b])

        def gather_dma(g):
            b = slot(g)
            return pltpu.make_async_copy(
                table_hbm.at[fb_v.at[b]], gb_v.at[b], sem_g.at[b])

        def out_dma(g):
            b = slot(g)
            return pltpu.make_async_copy(
                ob_v.at[b],
                out_hbm.at[pl.ds(obase + g * (CHUNK // 2), CHUNK // 2)],
                sem_o.at[b])

        def launch_gather(g):
            b = slot(g)
            xoff = lax.rem(xbase + g, 8)
            for s in range(CHUNK // L):
                sl = pl.ds(s * L, L)
                fb_v[b, sl] = xb_v[b, xoff, sl] >> 1
            gather_dma(g).start()

        x_dma(0).start()
        x_dma(1).start()
        x_dma(0).wait()
        launch_gather(0)

        def chunk_body(g, carry):
            b = slot(g)
            xoff = lax.rem(xbase + g, 8)

            @pl.when(g + 1 < n_chunks)
            def _():
                @pl.when(g + 1 >= NBUF)
                def _():
                    out_dma(g + 1 - NBUF).wait()
                x_dma(g + 1).wait()
                launch_gather(g + 1)

            @pl.when(g + 2 < n_chunks)
            def _():
                x_dma(g + 2).start()

            gather_dma(g).wait()

            def grp(t, c):
                idxv = xb_v[b, xoff, pl.ds(t * L, L)]
                for r in range(L):
                    row = t * L + r
                    half = idxv[r] & 1
                    cb = half * D
                    vs = [gb_v[b, row, pl.ds(cb + q * L, L)]
                          for q in range(D // L)]
                    acc = None
                    for v in vs:
                        acc = v * v if acc is None else acc + v * v
                    sv = jnp.full((L,), jnp.sum(acc), jnp.float32)
                    scale = _rsqrt(sv)
                    orow = t * (L // 2) + r // 2
                    oc = (r % 2) * D
                    for q, v in enumerate(vs):
                        ob_v[b, orow, pl.ds(oc + q * L, L)] = v * scale
                return c

            lax.fori_loop(0, CHUNK // L, grp, 0)
            out_dma(g).start()
            return carry

        lax.fori_loop(0, n_chunks, chunk_body, 0)
        for t in range(NBUF):
            out_dma(n_chunks - 1 - t).wait()

    return body(x2, table2)


def kernel(x, table):
    b, h = x.shape
    n = b * h
    x2 = x.reshape(n // 128, 128)
    table2 = table.reshape(table.shape[0] // 2, 128)
    out = _embed_normalize(x2, table2, n)
    return out.reshape(b, h, D)


# speedup vs baseline: 1.2828x; 1.2828x over previous
import functools

import jax
import jax.numpy as jnp
from jax import lax
from jax.experimental import pallas as pl
from jax.experimental.pallas import tpu as pltpu
from jax.experimental.pallas import tpu_sc as plsc

D = 64
L = 16
BBLK = 128
NBUF = 3


def _rsqrt(x):
    i = plsc.bitcast(x, jnp.int32)
    i = jnp.int32(0x5F3759DF) - (i >> 1)
    y = plsc.bitcast(i, jnp.float32)
    h = x * jnp.float32(0.5)
    for _ in range(2):
        y = y * (jnp.float32(1.5) - h * y * y)
    return y


@functools.partial(jax.jit, static_argnames=("hist", "batch"))
def _embed_normalize(xt, table_p, hist, batch):
    info = plsc.get_sparse_core_info()
    nc, ns = info.num_cores, info.num_subcores
    mesh = plsc.VectorSubcoreMesh(core_axis_name="c", subcore_axis_name="s")

    @functools.partial(
        pl.kernel,
        mesh=mesh,
        out_type=jax.ShapeDtypeStruct((hist, D, batch), jnp.float32),
        compiler_params=pltpu.CompilerParams(needs_layout_passes=False),
        scratch_types=[
            pltpu.VMEM((hist, BBLK), jnp.int32),
            pltpu.VMEM((NBUF, BBLK, 2 * D), jnp.float32),
            pltpu.VMEM((NBUF, D, BBLK), jnp.float32),
            pltpu.SemaphoreType.DMA((NBUF,)),
            pltpu.SemaphoreType.DMA((NBUF,)),
        ],
    )
    def body(x_hbm, table_hbm, out_hbm, xb_v, gb_v, ob_v, sem_g, sem_o):
        wid = lax.axis_index("s") * nc + lax.axis_index("c")
        bcol = wid * BBLK

        def slot(g):
            return lax.rem(g, NBUF)

        def gather_dma(g):
            b = slot(g)
            return pltpu.make_async_copy(
                table_hbm.at[xb_v.at[g]], gb_v.at[b], sem_g.at[b])

        def out_dma(g):
            b = slot(g)
            return pltpu.make_async_copy(
                ob_v.at[b],
                out_hbm.at[g, :, pl.ds(bcol, BBLK)], sem_o.at[b])

        pltpu.sync_copy(x_hbm.at[:, pl.ds(bcol, BBLK)], xb_v)
        gather_dma(0).start()

        lanes = lax.iota(jnp.int32, L)

        def chunk_body(g, carry):
            b = slot(g)

            @pl.when(g + 1 < hist)
            def _():
                gather_dma(g + 1).start()

            gather_dma(g).wait()

            @pl.when(g >= NBUF)
            def _():
                out_dma(g - NBUF).wait()

            gb = gb_v.at[b]
            ob = ob_v.at[b]

            def grp(t, c):
                rowv = t * L + lanes
                ss = jnp.zeros((L,), jnp.float32)
                rv = lanes
                for d in range(D):
                    v = plsc.load_gather(gb, [rowv, rv])
                    ss = ss + v * v
                    rv = rv + 1
                    rv = jnp.where(rv >= D, rv - D, rv)
                scale = _rsqrt(ss)
                rv = lanes
                for d in range(D):
                    v = plsc.load_gather(gb, [rowv, rv])
                    plsc.store_scatter(ob, [rv, rowv], v * scale)
                    rv = rv + 1
                    rv = jnp.where(rv >= D, rv - D, rv)
                return c

            lax.fori_loop(0, BBLK // L, grp, 0)
            out_dma(g).start()
            return carry

        lax.fori_loop(0, hist, chunk_body, 0)
        for t in range(NBUF):
            out_dma(hist - 1 - t).wait()

    return body(xt, table_p)


def kernel(x, table):
    b, h = x.shape
    table_p = jnp.pad(table, ((0, 0), (0, table.shape[1])))
    out = _embed_normalize(x.T, table_p, h, b)
    return out.transpose(2, 0, 1)
